# Initial kernel scaffold; baseline (speedup 1.0000x reference)
import jax
import jax.numpy as jnp
from jax.experimental import pallas as pl


def kernel(x, edge_index, W_emb, b_emb, W1_rel, W1_root, b1, W2_rel, W2_root, b2, Wf, bf):
    raise NotImplementedError("write your pallas kernel here")


# speedup vs baseline: 4.0198x; 4.0198x over previous
import functools

import jax
import jax.numpy as jnp
from jax import lax
from jax.experimental import pallas as pl
from jax.experimental.pallas import tpu as pltpu
from jax.experimental.pallas import tpu_sc as plsc

_N = 50000
_E = 800000
_IN_DIM = 128
_EMBED = 64
_OUT = 64
_HALF = 32

_NUM_TILES = 16
_CHUNK = 128
_NCHUNKS = _E // _CHUNK
_BASE_ITERS = _NCHUNKS // _NUM_TILES
_EXTRA_TILES = _NCHUNKS % _NUM_TILES
_WB = 3128
_WB_LAST = _N - 15 * _WB

_CNT_ROWS = 25088
_TRASH = 25000
_CNT_W = 8
_HALF_N = _N // 2
_CWB = 3128
_CWB_LAST = _HALF_N - 7 * _CWB

_TCB = 2000


def _tc1_body(x_ref, we_ref, be_ref, wrel_ref, wroot_ref, b1_ref,
              mlo_ref, mhi_ref, r1_ref):
    h0 = jnp.dot(x_ref[...], we_ref[...],
                 preferred_element_type=jnp.float32) + be_ref[...]
    m1 = jnp.dot(h0, wrel_ref[...], preferred_element_type=jnp.float32)
    mlo_ref[...] = m1[:, :_HALF]
    mhi_ref[...] = m1[:, _HALF:]
    r1_ref[...] = jnp.dot(h0, wroot_ref[...],
                          preferred_element_type=jnp.float32) + b1_ref[...]


def _tc_stage1(x, W_emb, b_emb2, W1_rel, W1_root, b12):
    full = lambda i: (0, 0)
    row = lambda i: (i, 0)
    return pl.pallas_call(
        _tc1_body,
        grid=(_N // _TCB,),
        in_specs=[
            pl.BlockSpec((_TCB, _IN_DIM), row),
            pl.BlockSpec((_IN_DIM, _EMBED), full),
            pl.BlockSpec((1, _EMBED), full),
            pl.BlockSpec((_EMBED, _EMBED), full),
            pl.BlockSpec((_EMBED, _EMBED), full),
            pl.BlockSpec((1, _EMBED), full),
        ],
        out_specs=[
            pl.BlockSpec((_TCB, _HALF), row),
            pl.BlockSpec((_TCB, _HALF), row),
            pl.BlockSpec((_TCB, _EMBED), row),
        ],
        out_shape=[
            jax.ShapeDtypeStruct((_N, _HALF), jnp.float32),
            jax.ShapeDtypeStruct((_N, _HALF), jnp.float32),
            jax.ShapeDtypeStruct((_N, _EMBED), jnp.float32),
        ],
    )(x, W_emb, b_emb2, W1_rel, W1_root, b12)


def _tc2_body(r1_ref, alo_ref, ahi_ref, cnt_ref, wrel_ref, wroot_ref, b2_ref,
              mlo_ref, mhi_ref, r2_ref):
    cnt = cnt_ref[...][:, 0:1]
    inv = 1.0 / jnp.maximum(cnt, 1.0)
    mean = jnp.concatenate([alo_ref[...], ahi_ref[...]], axis=1) * inv
    h1 = jnp.maximum(r1_ref[...] + mean, 0.0)
    m2 = jnp.dot(h1, wrel_ref[...], preferred_element_type=jnp.float32)
    mlo_ref[...] = m2[:, :_HALF]
    mhi_ref[...] = m2[:, _HALF:]
    r2_ref[...] = jnp.dot(h1, wroot_ref[...],
                          preferred_element_type=jnp.float32) + b2_ref[...]


def _tc_stage2(r1, alo, ahi, cnt, W2_rel, W2_root, b22):
    full = lambda i: (0, 0)
    row = lambda i: (i, 0)
    return pl.pallas_call(
        _tc2_body,
        grid=(_N // _TCB,),
        in_specs=[
            pl.BlockSpec((_TCB, _EMBED), row),
            pl.BlockSpec((_TCB, _HALF), row),
            pl.BlockSpec((_TCB, _HALF), row),
            pl.BlockSpec((_TCB, _CNT_W), row),
            pl.BlockSpec((_EMBED, _EMBED), full),
            pl.BlockSpec((_EMBED, _EMBED), full),
            pl.BlockSpec((1, _EMBED), full),
        ],
        out_specs=[
            pl.BlockSpec((_TCB, _HALF), row),
            pl.BlockSpec((_TCB, _HALF), row),
            pl.BlockSpec((_TCB, _EMBED), row),
        ],
        out_shape=[
            jax.ShapeDtypeStruct((_N, _HALF), jnp.float32),
            jax.ShapeDtypeStruct((_N, _HALF), jnp.float32),
            jax.ShapeDtypeStruct((_N, _EMBED), jnp.float32),
        ],
    )(r1, alo, ahi, cnt, W2_rel, W2_root, b22)


def _tc3_body(r2_ref, alo_ref, ahi_ref, cnt_ref, wf_ref, bf_ref, out_ref):
    cnt = cnt_ref[...][:, 0:1]
    inv = 1.0 / jnp.maximum(cnt, 1.0)
    mean = jnp.concatenate([alo_ref[...], ahi_ref[...]], axis=1) * inv
    h2 = jnp.maximum(r2_ref[...] + mean, 0.0)
    out_ref[...] = jnp.dot(h2, wf_ref[...],
                           preferred_element_type=jnp.float32) + bf_ref[...]


def _tc_stage3(r2, alo, ahi, cnt, Wf, bf2):
    full = lambda i: (0, 0)
    row = lambda i: (i, 0)
    return pl.pallas_call(
        _tc3_body,
        grid=(_N // _TCB,),
        in_specs=[
            pl.BlockSpec((_TCB, _EMBED), row),
            pl.BlockSpec((_TCB, _HALF), row),
            pl.BlockSpec((_TCB, _HALF), row),
            pl.BlockSpec((_TCB, _CNT_W), row),
            pl.BlockSpec((_EMBED, _OUT), full),
            pl.BlockSpec((1, _OUT), full),
        ],
        out_specs=[pl.BlockSpec((_TCB, _OUT), row)],
        out_shape=[jax.ShapeDtypeStruct((_N, _OUT), jnp.float32)],
    )(r2, alo, ahi, cnt, Wf, bf2)


def _make_sc_agg(with_counts):
    mesh = plsc.VectorSubcoreMesh(core_axis_name="c", subcore_axis_name="s",
                                  num_cores=2, num_subcores=_NUM_TILES)

    outs = [
        jax.ShapeDtypeStruct((_N, _HALF), jnp.float32),
        jax.ShapeDtypeStruct((_N, _HALF), jnp.float32),
    ]
    scratch = [
        pltpu.VMEM((_CHUNK,), jnp.int32),
        pltpu.VMEM((_CHUNK,), jnp.int32),
        pltpu.VMEM((_CHUNK, _HALF), jnp.float32),
        pltpu.VMEM_SHARED((_N, _HALF), jnp.float32),
    ]
    if with_counts:
        outs.append(jax.ShapeDtypeStruct((_N, _CNT_W), jnp.float32))
        scratch += [
            pltpu.VMEM((_CHUNK,), jnp.int32),
            pltpu.VMEM((_CHUNK, _CNT_W), jnp.float32),
            pltpu.VMEM_SHARED((_CNT_ROWS, _CNT_W), jnp.float32),
        ]
    scratch.append(pltpu.SemaphoreType.DMA)

    def body(m_lo, m_hi, src_h, dst_h, zeros_h, zcnt_h, ones_h, *rest):
        if with_counts:
            (out_lo, out_hi, out_cnt, src_v, dst_v, rows_v, acc_sh,
             cidx_v, ones_v, cnt_sh, sem) = rest
        else:
            (out_lo, out_hi, src_v, dst_v, rows_v, acc_sh, sem) = rest

        c = lax.axis_index("c")
        s = lax.axis_index("s")

        @pl.when(s < 15)
        def _():
            pltpu.sync_copy(zeros_h, acc_sh.at[pl.ds(s * _WB, _WB), :])

        @pl.when(s == 15)
        def _():
            pltpu.sync_copy(zeros_h.at[pl.ds(0, _WB_LAST), :],
                            acc_sh.at[pl.ds(15 * _WB, _WB_LAST), :])

        if with_counts:
            pltpu.sync_copy(zcnt_h, cnt_sh.at[pl.ds(s * 1568, 1568), :])
            pltpu.sync_copy(ones_h, ones_v)
        plsc.subcore_barrier()

        def edge_phase(mslab, lo):
            iters = _BASE_ITERS + jnp.where(s < _EXTRA_TILES, 1, 0)

            def chunk(i, carry):
                b = (s + _NUM_TILES * i) * _CHUNK
                pltpu.sync_copy(src_h.at[pl.ds(b, _CHUNK)], src_v)
                pltpu.sync_copy(dst_h.at[pl.ds(b, _CHUNK)], dst_v)
                pltpu.async_copy(mslab.at[src_v], rows_v, sem).wait()
                if with_counts:
                    for k in range(_CHUNK // 16):
                        d = dst_v[pl.ds(k * 16, 16)]
                        ok = (d >= lo) & (d < lo + _HALF_N)
                        cidx_v[pl.ds(k * 16, 16)] = jnp.where(ok, d - lo,
                                                              _TRASH)
                    pltpu.sync_copy(ones_v, cnt_sh.at[cidx_v], add=True)
                pltpu.sync_copy(rows_v, acc_sh.at[dst_v], add=True)
                return carry

            lax.fori_loop(0, iters, chunk, 0)

        @pl.when(c == 0)
        def _():
            edge_phase(m_lo, 0)

        @pl.when(c == 1)
        def _():
            edge_phase(m_hi, _HALF_N)

        plsc.subcore_barrier()

        def wb_phase(oslab, lo):
            @pl.when(s < 15)
            def _():
                pltpu.sync_copy(acc_sh.at[pl.ds(s * _WB, _WB), :],
                                oslab.at[pl.ds(s * _WB, _WB), :])

            @pl.when(s == 15)
            def _():
                pltpu.sync_copy(acc_sh.at[pl.ds(15 * _WB, _WB_LAST), :],
                                oslab.at[pl.ds(15 * _WB, _WB_LAST), :])

            if with_counts:
                @pl.when(s < 7)
                def _():
                    pltpu.sync_copy(
                        cnt_sh.at[pl.ds(s * _CWB, _CWB), :],
                        out_cnt.at[pl.ds(lo + s * _CWB, _CWB), :])

                @pl.when(s == 7)
                def _():
                    pltpu.sync_copy(
                        cnt_sh.at[pl.ds(7 * _CWB, _CWB_LAST), :],
                        out_cnt.at[pl.ds(lo + 7 * _CWB, _CWB_LAST), :])

        @pl.when(c == 0)
        def _():
            wb_phase(out_lo, 0)

        @pl.when(c == 1)
        def _():
            wb_phase(out_hi, _HALF_N)

    return pl.kernel(body, out_type=tuple(outs), mesh=mesh,
                     scratch_types=tuple(scratch),
                     compiler_params=pltpu.CompilerParams(
                         use_tc_tiling_on_sc=False))


_sc_agg_counts = _make_sc_agg(True)
_sc_agg = _make_sc_agg(False)


def kernel(x, edge_index, W_emb, b_emb, W1_rel, W1_root, b1,
           W2_rel, W2_root, b2, Wf, bf):
    src = edge_index[0]
    dst = edge_index[1]
    b_emb2 = b_emb.reshape(1, _EMBED)
    b12 = b1.reshape(1, _EMBED)
    b22 = b2.reshape(1, _EMBED)
    bf2 = bf.reshape(1, _OUT)

    zeros_h = jnp.zeros((_WB, _HALF), jnp.float32)
    zcnt_h = jnp.zeros((1568, _CNT_W), jnp.float32)
    ones_h = jnp.ones((_CHUNK, _CNT_W), jnp.float32)

    m1_lo, m1_hi, r1 = _tc_stage1(x, W_emb, b_emb2, W1_rel, W1_root, b12)
    a1_lo, a1_hi, cnt = _sc_agg_counts(m1_lo, m1_hi, src, dst,
                                       zeros_h, zcnt_h, ones_h)
    m2_lo, m2_hi, r2 = _tc_stage2(r1, a1_lo, a1_hi, cnt, W2_rel, W2_root, b22)
    a2_lo, a2_hi = _sc_agg(m2_lo, m2_hi, src, dst, zeros_h, zcnt_h, ones_h)
    (out,) = _tc_stage3(r2, a2_lo, a2_hi, cnt, Wf, bf2)
    return out


# speedup vs baseline: 6.6751x; 1.6606x over previous
import functools

import jax
import jax.numpy as jnp
from jax import lax
from jax.experimental import pallas as pl
from jax.experimental.pallas import tpu as pltpu
from jax.experimental.pallas import tpu_sc as plsc

_N = 50000
_E = 800000
_IN_DIM = 128
_EMBED = 64
_OUT = 64
_HALF = 32

_NUM_TILES = 16
_CHUNK = 128
_NCHUNKS = _E // _CHUNK
_BASE_ITERS = _NCHUNKS // _NUM_TILES
_EXTRA_TILES = _NCHUNKS % _NUM_TILES
_WB = 3128
_WB_LAST = _N - 15 * _WB

_CNT_ROWS = 25088
_TRASH = 25000
_CNT_W = 8
_HALF_N = _N // 2
_CWB = 3128
_CWB_LAST = _HALF_N - 7 * _CWB

_TCB = 2000


def _tc1_body(x_ref, we_ref, be_ref, wrel_ref, wroot_ref, b1_ref,
              mlo_ref, mhi_ref, r1_ref):
    h0 = jnp.dot(x_ref[...], we_ref[...],
                 preferred_element_type=jnp.float32) + be_ref[...]
    m1 = jnp.dot(h0, wrel_ref[...], preferred_element_type=jnp.float32)
    mlo_ref[...] = m1[:, :_HALF]
    mhi_ref[...] = m1[:, _HALF:]
    r1_ref[...] = jnp.dot(h0, wroot_ref[...],
                          preferred_element_type=jnp.float32) + b1_ref[...]


def _tc_stage1(x, W_emb, b_emb2, W1_rel, W1_root, b12):
    full = lambda i: (0, 0)
    row = lambda i: (i, 0)
    return pl.pallas_call(
        _tc1_body,
        grid=(_N // _TCB,),
        in_specs=[
            pl.BlockSpec((_TCB, _IN_DIM), row),
            pl.BlockSpec((_IN_DIM, _EMBED), full),
            pl.BlockSpec((1, _EMBED), full),
            pl.BlockSpec((_EMBED, _EMBED), full),
            pl.BlockSpec((_EMBED, _EMBED), full),
            pl.BlockSpec((1, _EMBED), full),
        ],
        out_specs=[
            pl.BlockSpec((_TCB, _HALF), row),
            pl.BlockSpec((_TCB, _HALF), row),
            pl.BlockSpec((_TCB, _EMBED), row),
        ],
        out_shape=[
            jax.ShapeDtypeStruct((_N, _HALF), jnp.float32),
            jax.ShapeDtypeStruct((_N, _HALF), jnp.float32),
            jax.ShapeDtypeStruct((_N, _EMBED), jnp.float32),
        ],
    )(x, W_emb, b_emb2, W1_rel, W1_root, b12)


def _tc2_body(r1_ref, alo_ref, ahi_ref, cnt_ref, wrel_ref, wroot_ref, b2_ref,
              mlo_ref, mhi_ref, r2_ref):
    cnt = cnt_ref[...][:, 0:1]
    inv = 1.0 / jnp.maximum(cnt, 1.0)
    mean = jnp.concatenate([alo_ref[...], ahi_ref[...]], axis=1) * inv
    h1 = jnp.maximum(r1_ref[...] + mean, 0.0)
    m2 = jnp.dot(h1, wrel_ref[...], preferred_element_type=jnp.float32)
    mlo_ref[...] = m2[:, :_HALF]
    mhi_ref[...] = m2[:, _HALF:]
    r2_ref[...] = jnp.dot(h1, wroot_ref[...],
                          preferred_element_type=jnp.float32) + b2_ref[...]


def _tc_stage2(r1, alo, ahi, cnt, W2_rel, W2_root, b22):
    full = lambda i: (0, 0)
    row = lambda i: (i, 0)
    return pl.pallas_call(
        _tc2_body,
        grid=(_N // _TCB,),
        in_specs=[
            pl.BlockSpec((_TCB, _EMBED), row),
            pl.BlockSpec((_TCB, _HALF), row),
            pl.BlockSpec((_TCB, _HALF), row),
            pl.BlockSpec((_TCB, _CNT_W), row),
            pl.BlockSpec((_EMBED, _EMBED), full),
            pl.BlockSpec((_EMBED, _EMBED), full),
            pl.BlockSpec((1, _EMBED), full),
        ],
        out_specs=[
            pl.BlockSpec((_TCB, _HALF), row),
            pl.BlockSpec((_TCB, _HALF), row),
            pl.BlockSpec((_TCB, _EMBED), row),
        ],
        out_shape=[
            jax.ShapeDtypeStruct((_N, _HALF), jnp.float32),
            jax.ShapeDtypeStruct((_N, _HALF), jnp.float32),
            jax.ShapeDtypeStruct((_N, _EMBED), jnp.float32),
        ],
    )(r1, alo, ahi, cnt, W2_rel, W2_root, b22)


def _tc3_body(r2_ref, alo_ref, ahi_ref, cnt_ref, wf_ref, bf_ref, out_ref):
    cnt = cnt_ref[...][:, 0:1]
    inv = 1.0 / jnp.maximum(cnt, 1.0)
    mean = jnp.concatenate([alo_ref[...], ahi_ref[...]], axis=1) * inv
    h2 = jnp.maximum(r2_ref[...] + mean, 0.0)
    out_ref[...] = jnp.dot(h2, wf_ref[...],
                           preferred_element_type=jnp.float32) + bf_ref[...]


def _tc_stage3(r2, alo, ahi, cnt, Wf, bf2):
    full = lambda i: (0, 0)
    row = lambda i: (i, 0)
    return pl.pallas_call(
        _tc3_body,
        grid=(_N // _TCB,),
        in_specs=[
            pl.BlockSpec((_TCB, _EMBED), row),
            pl.BlockSpec((_TCB, _HALF), row),
            pl.BlockSpec((_TCB, _HALF), row),
            pl.BlockSpec((_TCB, _CNT_W), row),
            pl.BlockSpec((_EMBED, _OUT), full),
            pl.BlockSpec((1, _OUT), full),
        ],
        out_specs=[pl.BlockSpec((_TCB, _OUT), row)],
        out_shape=[jax.ShapeDtypeStruct((_N, _OUT), jnp.float32)],
    )(r2, alo, ahi, cnt, Wf, bf2)


def _make_sc_agg(with_counts):
    mesh = plsc.VectorSubcoreMesh(core_axis_name="c", subcore_axis_name="s",
                                  num_cores=2, num_subcores=_NUM_TILES)

    outs = [
        jax.ShapeDtypeStruct((_N, _HALF), jnp.float32),
        jax.ShapeDtypeStruct((_N, _HALF), jnp.float32),
    ]
    scratch = [
        pltpu.VMEM((2, _CHUNK), jnp.int32),
        pltpu.VMEM((2, _CHUNK), jnp.int32),
        pltpu.VMEM((_CHUNK, _HALF), jnp.float32),
        pltpu.VMEM((_CHUNK, _HALF), jnp.float32),
        pltpu.VMEM_SHARED((_N, _HALF), jnp.float32),
    ]
    if with_counts:
        outs.append(jax.ShapeDtypeStruct((_N, _CNT_W), jnp.float32))
        scratch += [
            pltpu.VMEM((_CHUNK,), jnp.int32),
            pltpu.VMEM((_CHUNK,), jnp.int32),
            pltpu.VMEM((_CHUNK, _CNT_W), jnp.float32),
            pltpu.VMEM_SHARED((_CNT_ROWS, _CNT_W), jnp.float32),
        ]
    scratch += [pltpu.SemaphoreType.DMA, pltpu.SemaphoreType.DMA,
                pltpu.SemaphoreType.DMA, pltpu.SemaphoreType.DMA]

    def body(m_lo, m_hi, edge_h, zeros_h, zcnt_h, ones_h, *rest):
        if with_counts:
            (out_lo, out_hi, out_cnt, idx0_v, idx1_v, rows0_v, rows1_v,
             acc_sh, cidx0_v, cidx1_v, ones_v, cnt_sh,
             semL0, semL1, semG0, semG1) = rest
            cidx = (cidx0_v, cidx1_v)
        else:
            (out_lo, out_hi, idx0_v, idx1_v, rows0_v, rows1_v, acc_sh,
             semL0, semL1, semG0, semG1) = rest
        idx = (idx0_v, idx1_v)
        rows = (rows0_v, rows1_v)
        semL = (semL0, semL1)
        semG = (semG0, semG1)

        c = lax.axis_index("c")
        s = lax.axis_index("s")

        @pl.when(s < 15)
        def _():
            pltpu.sync_copy(zeros_h, acc_sh.at[pl.ds(s * _WB, _WB), :])

        @pl.when(s == 15)
        def _():
            pltpu.sync_copy(zeros_h.at[pl.ds(0, _WB_LAST), :],
                            acc_sh.at[pl.ds(15 * _WB, _WB_LAST), :])

        if with_counts:
            pltpu.sync_copy(zcnt_h, cnt_sh.at[pl.ds(s * 1568, 1568), :])
            pltpu.sync_copy(ones_h, ones_v)
        plsc.subcore_barrier()

        def edge_phase(mslab, lo):
            max_off = (_NCHUNKS - 1) * _CHUNK

            def issue_l(b, i):
                off = jnp.minimum((s + _NUM_TILES * i) * _CHUNK, max_off)
                pltpu.async_copy(edge_h.at[:, pl.ds(off, _CHUNK)],
                                 idx[b], semL[b])

            def wait_l(b):
                pltpu.make_async_copy(edge_h.at[:, pl.ds(0, _CHUNK)],
                                      idx[b], semL[b]).wait()

            def issue_g(b):
                pltpu.async_copy(mslab.at[idx[b].at[0]], rows[b], semG[b])

            def wait_g(b):
                pltpu.make_async_copy(mslab.at[idx[b].at[0]],
                                      rows[b], semG[b]).wait()

            def do_counts(b):
                drow = idx[b].at[1]
                for k in range(_CHUNK // 16):
                    d = drow[pl.ds(k * 16, 16)]
                    ok = (d >= lo) & (d < lo + _HALF_N)
                    cidx[b][pl.ds(k * 16, 16)] = jnp.where(ok, d - lo, _TRASH)
                pltpu.sync_copy(ones_v, cnt_sh.at[cidx[b]], add=True)

            def do_s(b):
                pltpu.sync_copy(rows[b], acc_sh.at[idx[b].at[1]], add=True)

            issue_l(0, 0)
            wait_l(0)
            issue_g(0)
            issue_l(1, 1)

            def pair(g, carry):
                wait_l(1)
                issue_g(1)
                wait_g(0)
                if with_counts:
                    do_counts(0)
                do_s(0)
                issue_l(0, 2 * g + 2)
                wait_g(1)
                if with_counts:
                    do_counts(1)
                do_s(1)
                issue_l(1, 2 * g + 3)
                wait_l(0)
                issue_g(0)
                return carry

            lax.fori_loop(0, _BASE_ITERS // 2, pair, 0)

            wait_l(1)
            wait_g(0)

            @pl.when(s < _EXTRA_TILES)
            def _():
                if with_counts:
                    do_counts(0)
                do_s(0)

        @pl.when(c == 0)
        def _():
            edge_phase(m_lo, 0)

        @pl.when(c == 1)
        def _():
            edge_phase(m_hi, _HALF_N)

        plsc.subcore_barrier()

        def wb_phase(oslab, lo):
            @pl.when(s < 15)
            def _():
                pltpu.sync_copy(acc_sh.at[pl.ds(s * _WB, _WB), :],
                                oslab.at[pl.ds(s * _WB, _WB), :])

            @pl.when(s == 15)
            def _():
                pltpu.sync_copy(acc_sh.at[pl.ds(15 * _WB, _WB_LAST), :],
                                oslab.at[pl.ds(15 * _WB, _WB_LAST), :])

            if with_counts:
                @pl.when(s < 7)
                def _():
                    pltpu.sync_copy(
                        cnt_sh.at[pl.ds(s * _CWB, _CWB), :],
                        out_cnt.at[pl.ds(lo + s * _CWB, _CWB), :])

                @pl.when(s == 7)
                def _():
                    pltpu.sync_copy(
                        cnt_sh.at[pl.ds(7 * _CWB, _CWB_LAST), :],
                        out_cnt.at[pl.ds(lo + 7 * _CWB, _CWB_LAST), :])

        @pl.when(c == 0)
        def _():
            wb_phase(out_lo, 0)

        @pl.when(c == 1)
        def _():
            wb_phase(out_hi, _HALF_N)

    return pl.kernel(body, out_type=tuple(outs), mesh=mesh,
                     scratch_types=tuple(scratch),
                     compiler_params=pltpu.CompilerParams(
                         use_tc_tiling_on_sc=False))


_sc_agg_counts = _make_sc_agg(True)
_sc_agg = _make_sc_agg(False)


def kernel(x, edge_index, W_emb, b_emb, W1_rel, W1_root, b1,
           W2_rel, W2_root, b2, Wf, bf):
    b_emb2 = b_emb.reshape(1, _EMBED)
    b12 = b1.reshape(1, _EMBED)
    b22 = b2.reshape(1, _EMBED)
    bf2 = bf.reshape(1, _OUT)

    zeros_h = jnp.zeros((_WB, _HALF), jnp.float32)
    zcnt_h = jnp.zeros((1568, _CNT_W), jnp.float32)
    ones_h = jnp.ones((_CHUNK, _CNT_W), jnp.float32)

    m1_lo, m1_hi, r1 = _tc_stage1(x, W_emb, b_emb2, W1_rel, W1_root, b12)
    a1_lo, a1_hi, cnt = _sc_agg_counts(m1_lo, m1_hi, edge_index,
                                       zeros_h, zcnt_h, ones_h)
    m2_lo, m2_hi, r2 = _tc_stage2(r1, a1_lo, a1_hi, cnt, W2_rel, W2_root, b22)
    a2_lo, a2_hi = _sc_agg(m2_lo, m2_hi, edge_index,
                           zeros_h, zcnt_h, ones_h)
    (out,) = _tc_stage3(r2, a2_lo, a2_hi, cnt, Wf, bf2)
    return out


# speedup vs baseline: 6.8489x; 1.0260x over previous
import jax
import jax.numpy as jnp
from jax import lax
from jax.experimental import pallas as pl
from jax.experimental.pallas import tpu as pltpu
from jax.experimental.pallas import tpu_sc as plsc

_N = 50000
_E = 800000
_IN_DIM = 128
_EMBED = 64
_OUT = 64
_HALF = 32

_NUM_TILES = 16
_CHUNK = 128
_NCHUNKS = _E // _CHUNK
_BASE_ITERS = _NCHUNKS // _NUM_TILES
_EXTRA_TILES = _NCHUNKS % _NUM_TILES
_WB = 3128
_WB_LAST = _N - 15 * _WB

_CNT_ROWS = 25088
_TRASH = 25000
_CNT_W = 8
_HALF_N = _N // 2
_CZ = _CNT_ROWS // _NUM_TILES
_CWB = 3128
_CWB_LAST = _HALF_N - 7 * _CWB

_TCB = 2000
_TCB3 = 2560
_TCG3 = 20


def _tc1_body(x_ref, we_ref, be_ref, wrel_ref, wroot_ref, b1_ref,
              mlo_ref, mhi_ref, r1_ref):
    h0 = jnp.dot(x_ref[...], we_ref[...],
                 preferred_element_type=jnp.float32) + be_ref[...]
    m1 = jnp.dot(h0, wrel_ref[...], preferred_element_type=jnp.float32)
    mlo_ref[...] = m1[:, :_HALF]
    mhi_ref[...] = m1[:, _HALF:]
    r1_ref[...] = jnp.dot(h0, wroot_ref[...],
                          preferred_element_type=jnp.float32) + b1_ref[...]


def _tc_stage1(x, W_emb, b_emb2, W1_rel, W1_root, b12):
    full = lambda i: (0, 0)
    row = lambda i: (i, 0)
    return pl.pallas_call(
        _tc1_body,
        grid=(_N // _TCB,),
        in_specs=[
            pl.BlockSpec((_TCB, _IN_DIM), row),
            pl.BlockSpec((_IN_DIM, _EMBED), full),
            pl.BlockSpec((1, _EMBED), full),
            pl.BlockSpec((_EMBED, _EMBED), full),
            pl.BlockSpec((_EMBED, _EMBED), full),
            pl.BlockSpec((1, _EMBED), full),
        ],
        out_specs=[
            pl.BlockSpec((_TCB, _HALF), row),
            pl.BlockSpec((_TCB, _HALF), row),
            pl.BlockSpec((_TCB, _EMBED), row),
        ],
        out_shape=[
            jax.ShapeDtypeStruct((_N, _HALF), jnp.float32),
            jax.ShapeDtypeStruct((_N, _HALF), jnp.float32),
            jax.ShapeDtypeStruct((_N, _EMBED), jnp.float32),
        ],
    )(x, W_emb, b_emb2, W1_rel, W1_root, b12)


def _tc2_body(r1_ref, alo_ref, ahi_ref, cnt_ref, wrel_ref, wroot_ref, b2_ref,
              mlo_ref, mhi_ref, r2_ref):
    cnt = cnt_ref[...][:, 0:1]
    inv = 1.0 / jnp.maximum(cnt, 1.0)
    mean = jnp.concatenate([alo_ref[...], ahi_ref[...]], axis=1) * inv
    h1 = jnp.maximum(r1_ref[...] + mean, 0.0)
    m2 = jnp.dot(h1, wrel_ref[...], preferred_element_type=jnp.float32)
    mlo_ref[...] = m2[:, :_HALF]
    mhi_ref[...] = m2[:, _HALF:]
    r2_ref[...] = jnp.dot(h1, wroot_ref[...],
                          preferred_element_type=jnp.float32) + b2_ref[...]


def _tc_stage2(r1, alo, ahi, cnt, W2_rel, W2_root, b22):
    full = lambda i: (0, 0)
    row = lambda i: (i, 0)
    return pl.pallas_call(
        _tc2_body,
        grid=(_N // _TCB,),
        in_specs=[
            pl.BlockSpec((_TCB, _EMBED), row),
            pl.BlockSpec((_TCB, _HALF), row),
            pl.BlockSpec((_TCB, _HALF), row),
            pl.BlockSpec((_TCB, _CNT_W), row),
            pl.BlockSpec((_EMBED, _EMBED), full),
            pl.BlockSpec((_EMBED, _EMBED), full),
            pl.BlockSpec((1, _EMBED), full),
        ],
        out_specs=[
            pl.BlockSpec((_TCB, _HALF), row),
            pl.BlockSpec((_TCB, _HALF), row),
            pl.BlockSpec((_TCB, _EMBED), row),
        ],
        out_shape=[
            jax.ShapeDtypeStruct((_N, _HALF), jnp.float32),
            jax.ShapeDtypeStruct((_N, _HALF), jnp.float32),
            jax.ShapeDtypeStruct((_N, _EMBED), jnp.float32),
        ],
    )(r1, alo, ahi, cnt, W2_rel, W2_root, b22)


def _tc3_body(r2_ref, alo_ref, ahi_ref, cnt_ref, wf_ref, bf_ref, out_ref):
    cnt = cnt_ref[...][:, 0:1]
    inv = 1.0 / jnp.maximum(cnt, 1.0)
    mean = jnp.concatenate([alo_ref[...], ahi_ref[...]], axis=1) * inv
    h2 = jnp.maximum(r2_ref[...] + mean, 0.0)
    outT = lax.dot_general(wf_ref[...], h2,
                           (((0,), (1,)), ((), ())),
                           preferred_element_type=jnp.float32)
    out_ref[...] = outT + bf_ref[...]


def _tc_stage3(r2, alo, ahi, cnt, Wf, bfT):
    full = lambda i: (0, 0)
    row = lambda i: (i, 0)
    col = lambda i: (0, i)
    return pl.pallas_call(
        _tc3_body,
        grid=(_TCG3,),
        in_specs=[
            pl.BlockSpec((_TCB3, _EMBED), row),
            pl.BlockSpec((_TCB3, _HALF), row),
            pl.BlockSpec((_TCB3, _HALF), row),
            pl.BlockSpec((_TCB3, _CNT_W), row),
            pl.BlockSpec((_EMBED, _OUT), full),
            pl.BlockSpec((_OUT, 1), full),
        ],
        out_specs=[pl.BlockSpec((_OUT, _TCB3), col)],
        out_shape=[jax.ShapeDtypeStruct((_OUT, _N), jnp.float32)],
    )(r2, alo, ahi, cnt, Wf, bfT)


def _make_sc_agg(with_counts):
    mesh = plsc.VectorSubcoreMesh(core_axis_name="c", subcore_axis_name="s",
                                  num_cores=2, num_subcores=_NUM_TILES)

    outs = [
        jax.ShapeDtypeStruct((_N, _HALF), jnp.float32),
        jax.ShapeDtypeStruct((_N, _HALF), jnp.float32),
    ]
    scratch = [
        pltpu.VMEM((2, _CHUNK), jnp.int32),
        pltpu.VMEM((2, _CHUNK), jnp.int32),
        pltpu.VMEM((_CHUNK, _HALF), jnp.float32),
        pltpu.VMEM((_CHUNK, _HALF), jnp.float32),
        pltpu.VMEM_SHARED((_N, _HALF), jnp.float32),
    ]
    if with_counts:
        outs.append(jax.ShapeDtypeStruct((_N, _CNT_W), jnp.float32))
        scratch += [
            pltpu.VMEM((_CHUNK,), jnp.int32),
            pltpu.VMEM((_CHUNK,), jnp.int32),
            pltpu.VMEM((_CHUNK, _CNT_W), jnp.float32),
            pltpu.VMEM_SHARED((_CNT_ROWS, _CNT_W), jnp.float32),
        ]
    scratch += [pltpu.SemaphoreType.DMA, pltpu.SemaphoreType.DMA,
                pltpu.SemaphoreType.DMA, pltpu.SemaphoreType.DMA]

    def body(m_lo, m_hi, edge_h, zeros_h, zcnt_h, ones_h, *rest):
        if with_counts:
            (out_lo, out_hi, out_cnt, idx0_v, idx1_v, rows0_v, rows1_v,
             acc_sh, cidx0_v, cidx1_v, ones_v, cnt_sh,
             semL0, semL1, semG0, semG1) = rest
            cidx = (cidx0_v, cidx1_v)
        else:
            (out_lo, out_hi, idx0_v, idx1_v, rows0_v, rows1_v, acc_sh,
             semL0, semL1, semG0, semG1) = rest
        idx = (idx0_v, idx1_v)
        rows = (rows0_v, rows1_v)
        semL = (semL0, semL1)
        semG = (semG0, semG1)

        c = lax.axis_index("c")
        s = lax.axis_index("s")

        @pl.when(s < 15)
        def _():
            pltpu.sync_copy(zeros_h, acc_sh.at[pl.ds(s * _WB, _WB), :])

        @pl.when(s == 15)
        def _():
            pltpu.sync_copy(zeros_h.at[pl.ds(0, _WB_LAST), :],
                            acc_sh.at[pl.ds(15 * _WB, _WB_LAST), :])

        if with_counts:
            pltpu.sync_copy(zcnt_h, cnt_sh.at[pl.ds(s * _CZ, _CZ), :])
            pltpu.sync_copy(ones_h, ones_v)
        plsc.subcore_barrier()

        def edge_phase(mslab, lo):
            max_off = (_NCHUNKS - 1) * _CHUNK

            def issue_l(b, i):
                off = jnp.minimum((s + _NUM_TILES * i) * _CHUNK, max_off)
                pltpu.async_copy(edge_h.at[:, pl.ds(off, _CHUNK)],
                                 idx[b], semL[b])

            def wait_l(b):
                pltpu.make_async_copy(edge_h.at[:, pl.ds(0, _CHUNK)],
                                      idx[b], semL[b]).wait()

            def issue_g(b):
                pltpu.async_copy(mslab.at[idx[b].at[0]], rows[b], semG[b])

            def wait_g(b):
                pltpu.make_async_copy(mslab.at[idx[b].at[0]],
                                      rows[b], semG[b]).wait()

            def do_counts(b):
                drow = idx[b].at[1]
                for k in range(_CHUNK // 16):
                    d = drow[pl.ds(k * 16, 16)]
                    ok = (d >= lo) & (d < lo + _HALF_N)
                    cidx[b][pl.ds(k * 16, 16)] = jnp.where(ok, d - lo, _TRASH)
                pltpu.sync_copy(ones_v, cnt_sh.at[cidx[b]], add=True)

            def do_s(b):
                pltpu.sync_copy(rows[b], acc_sh.at[idx[b].at[1]], add=True)

            issue_l(0, 0)
            wait_l(0)
            issue_g(0)
            issue_l(1, 1)

            def pair(g, carry):
                wait_l(1)
                issue_g(1)
                wait_g(0)
                if with_counts:
                    do_counts(0)
                do_s(0)
                issue_l(0, 2 * g + 2)
                wait_g(1)
                if with_counts:
                    do_counts(1)
                do_s(1)
                issue_l(1, 2 * g + 3)
                wait_l(0)
                issue_g(0)
                return carry

            lax.fori_loop(0, _BASE_ITERS // 2, pair, 0)

            wait_l(1)
            wait_g(0)

            @pl.when(s < _EXTRA_TILES)
            def _():
                if with_counts:
                    do_counts(0)
                do_s(0)

        @pl.when(c == 0)
        def _():
            edge_phase(m_lo, 0)

        @pl.when(c == 1)
        def _():
            edge_phase(m_hi, _HALF_N)

        plsc.subcore_barrier()

        def wb_phase(oslab):
            @pl.when(s < 15)
            def _():
                pltpu.sync_copy(acc_sh.at[pl.ds(s * _WB, _WB), :],
                                oslab.at[pl.ds(s * _WB, _WB), :])

            @pl.when(s == 15)
            def _():
                pltpu.sync_copy(acc_sh.at[pl.ds(15 * _WB, _WB_LAST), :],
                                oslab.at[pl.ds(15 * _WB, _WB_LAST), :])

        def cnt_wb(lo):
            @pl.when(s < 7)
            def _():
                pltpu.sync_copy(cnt_sh.at[pl.ds(s * _CWB, _CWB), :],
                                out_cnt.at[pl.ds(lo + s * _CWB, _CWB), :])

            @pl.when(s == 7)
            def _():
                pltpu.sync_copy(
                    cnt_sh.at[pl.ds(7 * _CWB, _CWB_LAST), :],
                    out_cnt.at[pl.ds(lo + 7 * _CWB, _CWB_LAST), :])

        @pl.when(c == 0)
        def _():
            wb_phase(out_lo)
            if with_counts:
                cnt_wb(0)

        @pl.when(c == 1)
        def _():
            wb_phase(out_hi)
            if with_counts:
                cnt_wb(_HALF_N)

    return pl.kernel(body, out_type=tuple(outs), mesh=mesh,
                     scratch_types=tuple(scratch),
                     compiler_params=pltpu.CompilerParams(
                         use_tc_tiling_on_sc=False))


_sc_agg_counts = _make_sc_agg(True)
_sc_agg = _make_sc_agg(False)


def kernel(x, edge_index, W_emb, b_emb, W1_rel, W1_root, b1,
           W2_rel, W2_root, b2, Wf, bf):
    b_emb2 = b_emb.reshape(1, _EMBED)
    b12 = b1.reshape(1, _EMBED)
    b22 = b2.reshape(1, _EMBED)
    bfT = bf.reshape(_OUT, 1)

    zeros_h = jnp.zeros((_WB, _HALF), jnp.float32)
    zcnt_h = jnp.zeros((_CZ, _CNT_W), jnp.float32)
    ones_h = jnp.ones((_CHUNK, _CNT_W), jnp.float32)

    m1_lo, m1_hi, r1 = _tc_stage1(x, W_emb, b_emb2, W1_rel, W1_root, b12)
    a1_lo, a1_hi, cnt = _sc_agg_counts(m1_lo, m1_hi, edge_index,
                                       zeros_h, zcnt_h, ones_h)
    m2_lo, m2_hi, r2 = _tc_stage2(r1, a1_lo, a1_hi, cnt, W2_rel, W2_root, b22)
    a2_lo, a2_hi = _sc_agg(m2_lo, m2_hi, edge_index,
                           zeros_h, zcnt_h, ones_h)
    (outT,) = _tc_stage3(r2, a2_lo, a2_hi, cnt, Wf, bfT)
    return outT.T


# speedup vs baseline: 7.0151x; 1.0243x over previous
import jax
import jax.numpy as jnp
from jax import lax
from jax.experimental import pallas as pl
from jax.experimental.pallas import tpu as pltpu
from jax.experimental.pallas import tpu_sc as plsc

_N = 50000
_E = 800000
_IN_DIM = 128
_EMBED = 64
_OUT = 64
_HALF = 32

_NUM_TILES = 16
_CHUNK = 128
_NCHUNKS = _E // _CHUNK
_BASE_ITERS = _NCHUNKS // _NUM_TILES
_EXTRA_TILES = _NCHUNKS % _NUM_TILES
_WB = 3128
_WB_LAST = _N - 15 * _WB

_CNT_ROWS = 25008
_TRASH = 25000
_CNT_W = 8
_HALF_N = _N // 2
_CZ = _CNT_ROWS // _NUM_TILES
_CWB = 3128
_CWB_LAST = _HALF_N - 7 * _CWB

_TCB = 2000
_TCB3 = 2560
_TCG3 = 20


def _tc1_body(x_ref, we_ref, be_ref, wrel_ref, wroot_ref, b1_ref,
              mlo_ref, mhi_ref, r1_ref):
    h0 = jnp.dot(x_ref[...], we_ref[...],
                 preferred_element_type=jnp.float32) + be_ref[...]
    m1 = jnp.dot(h0, wrel_ref[...], preferred_element_type=jnp.float32)
    mlo_ref[...] = m1[:, :_HALF]
    mhi_ref[...] = m1[:, _HALF:]
    r1_ref[...] = jnp.dot(h0, wroot_ref[...],
                          preferred_element_type=jnp.float32) + b1_ref[...]


def _tc_stage1(x, W_emb, b_emb2, W1_rel, W1_root, b12):
    full = lambda i: (0, 0)
    row = lambda i: (i, 0)
    return pl.pallas_call(
        _tc1_body,
        grid=(_N // _TCB,),
        in_specs=[
            pl.BlockSpec((_TCB, _IN_DIM), row),
            pl.BlockSpec((_IN_DIM, _EMBED), full),
            pl.BlockSpec((1, _EMBED), full),
            pl.BlockSpec((_EMBED, _EMBED), full),
            pl.BlockSpec((_EMBED, _EMBED), full),
            pl.BlockSpec((1, _EMBED), full),
        ],
        out_specs=[
            pl.BlockSpec((_TCB, _HALF), row),
            pl.BlockSpec((_TCB, _HALF), row),
            pl.BlockSpec((_TCB, _EMBED), row),
        ],
        out_shape=[
            jax.ShapeDtypeStruct((_N, _HALF), jnp.float32),
            jax.ShapeDtypeStruct((_N, _HALF), jnp.float32),
            jax.ShapeDtypeStruct((_N, _EMBED), jnp.float32),
        ],
    )(x, W_emb, b_emb2, W1_rel, W1_root, b12)


def _tc2_body(r1_ref, alo_ref, ahi_ref, cnt_ref, wrel_ref, wroot_ref, b2_ref,
              mlo_ref, mhi_ref, r2_ref):
    cnt = cnt_ref[...][:, 0:1]
    inv = 1.0 / jnp.maximum(cnt, 1.0)
    mean = jnp.concatenate([alo_ref[...], ahi_ref[...]], axis=1) * inv
    h1 = jnp.maximum(r1_ref[...] + mean, 0.0)
    m2 = jnp.dot(h1, wrel_ref[...], preferred_element_type=jnp.float32)
    mlo_ref[...] = m2[:, :_HALF]
    mhi_ref[...] = m2[:, _HALF:]
    r2_ref[...] = jnp.dot(h1, wroot_ref[...],
                          preferred_element_type=jnp.float32) + b2_ref[...]


def _tc_stage2(r1, alo, ahi, cnt, W2_rel, W2_root, b22):
    full = lambda i: (0, 0)
    row = lambda i: (i, 0)
    return pl.pallas_call(
        _tc2_body,
        grid=(_N // _TCB,),
        in_specs=[
            pl.BlockSpec((_TCB, _EMBED), row),
            pl.BlockSpec((_TCB, _HALF), row),
            pl.BlockSpec((_TCB, _HALF), row),
            pl.BlockSpec((_TCB, _CNT_W), row),
            pl.BlockSpec((_EMBED, _EMBED), full),
            pl.BlockSpec((_EMBED, _EMBED), full),
            pl.BlockSpec((1, _EMBED), full),
        ],
        out_specs=[
            pl.BlockSpec((_TCB, _HALF), row),
            pl.BlockSpec((_TCB, _HALF), row),
            pl.BlockSpec((_TCB, _EMBED), row),
        ],
        out_shape=[
            jax.ShapeDtypeStruct((_N, _HALF), jnp.float32),
            jax.ShapeDtypeStruct((_N, _HALF), jnp.float32),
            jax.ShapeDtypeStruct((_N, _EMBED), jnp.float32),
        ],
    )(r1, alo, ahi, cnt, W2_rel, W2_root, b22)


def _tc3_body(r2_ref, alo_ref, ahi_ref, cnt_ref, wf_ref, bf_ref, out_ref):
    cnt = cnt_ref[...][:, 0:1]
    inv = 1.0 / jnp.maximum(cnt, 1.0)
    mean = jnp.concatenate([alo_ref[...], ahi_ref[...]], axis=1) * inv
    h2 = jnp.maximum(r2_ref[...] + mean, 0.0)
    outT = lax.dot_general(wf_ref[...], h2,
                           (((0,), (1,)), ((), ())),
                           preferred_element_type=jnp.float32)
    out_ref[...] = outT + bf_ref[...]


def _tc_stage3(r2, alo, ahi, cnt, Wf, bfT):
    full = lambda i: (0, 0)
    row = lambda i: (i, 0)
    col = lambda i: (0, i)
    return pl.pallas_call(
        _tc3_body,
        grid=(_TCG3,),
        in_specs=[
            pl.BlockSpec((_TCB3, _EMBED), row),
            pl.BlockSpec((_TCB3, _HALF), row),
            pl.BlockSpec((_TCB3, _HALF), row),
            pl.BlockSpec((_TCB3, _CNT_W), row),
            pl.BlockSpec((_EMBED, _OUT), full),
            pl.BlockSpec((_OUT, 1), full),
        ],
        out_specs=[pl.BlockSpec((_OUT, _TCB3), col)],
        out_shape=[jax.ShapeDtypeStruct((_OUT, _N), jnp.float32)],
    )(r2, alo, ahi, cnt, Wf, bfT)


def _make_sc_agg(with_counts):
    mesh = plsc.VectorSubcoreMesh(core_axis_name="c", subcore_axis_name="s",
                                  num_cores=2, num_subcores=_NUM_TILES)

    outs = [
        jax.ShapeDtypeStruct((_N, _HALF), jnp.float32),
        jax.ShapeDtypeStruct((_N, _HALF), jnp.float32),
    ]
    nbuf = 3
    scratch = []
    scratch += [pltpu.VMEM((2, _CHUNK), jnp.int32) for _ in range(nbuf)]
    scratch += [pltpu.VMEM((_CHUNK, _HALF), jnp.float32)
                for _ in range(nbuf)]
    scratch.append(pltpu.VMEM_SHARED((_N, _HALF), jnp.float32))
    if with_counts:
        outs.append(jax.ShapeDtypeStruct((_N, _CNT_W), jnp.float32))
        scratch += [pltpu.VMEM((_CHUNK,), jnp.int32) for _ in range(nbuf)]
        scratch += [
            pltpu.VMEM((_CHUNK, _CNT_W), jnp.float32),
            pltpu.VMEM_SHARED((_CNT_ROWS, _CNT_W), jnp.float32),
        ]
    scratch += [pltpu.SemaphoreType.DMA] * (3 * nbuf)

    def body(m_lo, m_hi, edge_h, zeros_h, zcnt_h, ones_h, *rest):
        if with_counts:
            (out_lo, out_hi, out_cnt, i0, i1, i2, r0, r1, r2,
             acc_sh, c0, c1, c2, ones_v, cnt_sh, *sems) = rest
            cidx = (c0, c1, c2)
        else:
            (out_lo, out_hi, i0, i1, i2, r0, r1, r2,
             acc_sh, *sems) = rest
        idx = (i0, i1, i2)
        rows = (r0, r1, r2)
        semL = tuple(sems[0:nbuf])
        semG = tuple(sems[nbuf:2 * nbuf])
        semS = tuple(sems[2 * nbuf:3 * nbuf])

        c = lax.axis_index("c")
        s = lax.axis_index("s")

        @pl.when(s < 15)
        def _():
            pltpu.sync_copy(zeros_h, acc_sh.at[pl.ds(s * _WB, _WB), :])

        @pl.when(s == 15)
        def _():
            pltpu.sync_copy(zeros_h.at[pl.ds(0, _WB_LAST), :],
                            acc_sh.at[pl.ds(15 * _WB, _WB_LAST), :])

        if with_counts:
            pltpu.sync_copy(zcnt_h, cnt_sh.at[pl.ds(s * _CZ, _CZ), :])
            pltpu.sync_copy(ones_h, ones_v)
        plsc.subcore_barrier()

        def edge_phase(mslab, lo):
            max_off = (_NCHUNKS - 1) * _CHUNK

            def issue_l(b, i):
                off = jnp.minimum((s + _NUM_TILES * i) * _CHUNK, max_off)
                pltpu.async_copy(edge_h.at[:, pl.ds(off, _CHUNK)],
                                 idx[b], semL[b])

            def wait_l(b):
                pltpu.make_async_copy(edge_h.at[:, pl.ds(0, _CHUNK)],
                                      idx[b], semL[b]).wait()

            def issue_g(b):
                pltpu.async_copy(mslab.at[idx[b].at[0]], rows[b], semG[b])

            def wait_g(b):
                pltpu.make_async_copy(mslab.at[idx[b].at[0]],
                                      rows[b], semG[b]).wait()

            def cidx_compute(b):
                drow = idx[b].at[1]
                for k in range(_CHUNK // 16):
                    d = drow[pl.ds(k * 16, 16)]
                    t = (d - lo).astype(jnp.uint32)
                    cidx[b][pl.ds(k * 16, 16)] = jnp.minimum(
                        t, jnp.uint32(_TRASH)).astype(jnp.int32)

            def issue_s(b):
                if with_counts:
                    cidx_compute(b)
                    pltpu.async_copy(ones_v, cnt_sh.at[cidx[b]], semS[b],
                                     add=True)
                pltpu.async_copy(rows[b], acc_sh.at[idx[b].at[1]], semS[b],
                                 add=True)

            def wait_s(b):
                if with_counts:
                    pltpu.make_async_copy(ones_v, cnt_sh.at[cidx[b]],
                                          semS[b]).wait()
                pltpu.make_async_copy(rows[b], acc_sh.at[idx[b].at[1]],
                                      semS[b]).wait()

            def sync_s(b):
                if with_counts:
                    cidx_compute(b)
                    pltpu.sync_copy(ones_v, cnt_sh.at[cidx[b]], add=True)
                pltpu.sync_copy(rows[b], acc_sh.at[idx[b].at[1]], add=True)

            issue_l(0, 0)
            wait_l(0)
            issue_g(0)
            issue_l(1, 1)
            wait_l(1)
            issue_g(1)
            wait_g(0)
            issue_s(0)
            issue_l(2, 2)

            def tri(g, carry):
                v0 = 2 + 3 * g
                for j in range(3):
                    b = (2 + j) % 3
                    bp = (1 + j) % 3
                    b2 = j % 3
                    wait_l(b)
                    issue_g(b)
                    wait_g(bp)
                    issue_s(bp)
                    wait_s(b2)
                    issue_l(b2, v0 + j + 1)
                return carry

            lax.fori_loop(0, (_BASE_ITERS - 3) // 3, tri, 0)

            wait_l(2)
            issue_g(2)
            wait_g(1)
            issue_s(1)
            wait_s(0)
            issue_l(0, 390)
            wait_g(2)
            issue_s(2)
            wait_s(1)
            wait_s(2)
            wait_l(0)

            @pl.when(s < _EXTRA_TILES)
            def _():
                pltpu.async_copy(mslab.at[idx[0].at[0]], rows[0],
                                 semG[0]).wait()
                sync_s(0)

        @pl.when(c == 0)
        def _():
            edge_phase(m_lo, 0)

        @pl.when(c == 1)
        def _():
            edge_phase(m_hi, _HALF_N)

        plsc.subcore_barrier()

        def wb_phase(oslab):
            @pl.when(s < 15)
            def _():
                pltpu.sync_copy(acc_sh.at[pl.ds(s * _WB, _WB), :],
                                oslab.at[pl.ds(s * _WB, _WB), :])

            @pl.when(s == 15)
            def _():
                pltpu.sync_copy(acc_sh.at[pl.ds(15 * _WB, _WB_LAST), :],
                                oslab.at[pl.ds(15 * _WB, _WB_LAST), :])

        def cnt_wb(lo):
            @pl.when(s < 7)
            def _():
                pltpu.sync_copy(cnt_sh.at[pl.ds(s * _CWB, _CWB), :],
                                out_cnt.at[pl.ds(lo + s * _CWB, _CWB), :])

            @pl.when(s == 7)
            def _():
                pltpu.sync_copy(
                    cnt_sh.at[pl.ds(7 * _CWB, _CWB_LAST), :],
                    out_cnt.at[pl.ds(lo + 7 * _CWB, _CWB_LAST), :])

        @pl.when(c == 0)
        def _():
            wb_phase(out_lo)
            if with_counts:
                cnt_wb(0)

        @pl.when(c == 1)
        def _():
            wb_phase(out_hi)
            if with_counts:
                cnt_wb(_HALF_N)

    return pl.kernel(body, out_type=tuple(outs), mesh=mesh,
                     scratch_types=tuple(scratch),
                     compiler_params=pltpu.CompilerParams(
                         use_tc_tiling_on_sc=False))


_sc_agg_counts = _make_sc_agg(True)
_sc_agg = _make_sc_agg(False)


def kernel(x, edge_index, W_emb, b_emb, W1_rel, W1_root, b1,
           W2_rel, W2_root, b2, Wf, bf):
    b_emb2 = b_emb.reshape(1, _EMBED)
    b12 = b1.reshape(1, _EMBED)
    b22 = b2.reshape(1, _EMBED)
    bfT = bf.reshape(_OUT, 1)

    zeros_h = jnp.zeros((_WB, _HALF), jnp.float32)
    zcnt_h = jnp.zeros((_CZ, _CNT_W), jnp.float32)
    ones_h = jnp.ones((_CHUNK, _CNT_W), jnp.float32)

    m1_lo, m1_hi, r1 = _tc_stage1(x, W_emb, b_emb2, W1_rel, W1_root, b12)
    a1_lo, a1_hi, cnt = _sc_agg_counts(m1_lo, m1_hi, edge_index,
                                       zeros_h, zcnt_h, ones_h)
    m2_lo, m2_hi, r2 = _tc_stage2(r1, a1_lo, a1_hi, cnt, W2_rel, W2_root, b22)
    a2_lo, a2_hi = _sc_agg(m2_lo, m2_hi, edge_index,
                           zeros_h, zcnt_h, ones_h)
    (outT,) = _tc_stage3(r2, a2_lo, a2_hi, cnt, Wf, bfT)
    return outT.T
